# Initial kernel scaffold; baseline (speedup 1.0000x reference)
import jax
import jax.numpy as jnp
from jax.experimental import pallas as pl


def kernel(lig_pos, prot_pos, prot_pos_Cb, prot_pos_C, prot_pos_O, prot_pos_N, cross_idx, W1, b1, W2, b2):
    raise NotImplementedError("write your pallas kernel here")


# speedup vs baseline: 5.7593x; 5.7593x over previous
import functools

import jax
import jax.numpy as jnp
from jax import lax
from jax.experimental import pallas as pl
from jax.experimental.pallas import tpu as pltpu
from jax.experimental.pallas import tpu_sc as plsc

N_PROT = 10000
N_LIG = 10000
E = 320000
RADIUS_EMB_DIM = 32
FOLD_DIM = 128
PROTEIN_RADIUS = 30.0

NC = 2
NS = 16
NW = NC * NS

EDGES_PER_TILE = E // NW
CHUNK = 2000
N_CHUNKS = EDGES_PER_TILE // CHUNK
IDX_MINOR = 125
IDX_ROWS_PER_CHUNK = CHUNK // IDX_MINOR
GROUPS = CHUNK // 16

FEAT = 5 * RADIUS_EMB_DIM
FEAT_PAD = 256
BLK = 1000

def _sc_body(p_hbm, ligx_hbm, ligy_hbm, ligz_hbm, src_hbm, dst2d_hbm, d2_hbm,
             ligx_v, ligy_v, ligz_v, src_v, dsti_v, prot_v, out_v, sem):
    wid = lax.axis_index("s") * NC + lax.axis_index("c")
    pltpu.sync_copy(ligx_hbm, ligx_v)
    pltpu.sync_copy(ligy_hbm, ligy_v)
    pltpu.sync_copy(ligz_hbm, ligz_v)

    def chunk_body(c, carry):
        row0 = wid * (EDGES_PER_TILE // IDX_MINOR) + c * IDX_ROWS_PER_CHUNK
        ebase = row0 * IDX_MINOR
        pltpu.sync_copy(src_hbm.at[pl.ds(ebase, CHUNK)], src_v)
        pltpu.sync_copy(dst2d_hbm.at[pl.ds(row0, IDX_ROWS_PER_CHUNK)], dsti_v)
        cps = []
        for j in range(IDX_ROWS_PER_CHUNK):
            cps.append(pltpu.async_copy(
                p_hbm.at[dsti_v.at[j]],
                prot_v.at[pl.ds(j * IDX_MINOR, IDX_MINOR)], sem))
        for cp in cps:
            cp.wait()

        def grp(g, carry2):
            off = g * 16
            rows = off + lax.iota(jnp.int32, 16)
            sv = src_v[pl.ds(off, 16)]
            lx = plsc.load_gather(ligx_v, [sv])
            ly = plsc.load_gather(ligy_v, [sv])
            lz = plsc.load_gather(ligz_v, [sv])
            for k in range(5):
                px = plsc.load_gather(prot_v, [rows, jnp.full((16,), 3 * k, jnp.int32)])
                py = plsc.load_gather(prot_v, [rows, jnp.full((16,), 3 * k + 1, jnp.int32)])
                pz = plsc.load_gather(prot_v, [rows, jnp.full((16,), 3 * k + 2, jnp.int32)])
                dx = lx - px
                dy = ly - py
                dz = lz - pz
                d2 = dx * dx + dy * dy + dz * dz
                plsc.store_scatter(out_v, [rows, jnp.full((16,), k, jnp.int32)], d2)
            return carry2

        lax.fori_loop(0, GROUPS, grp, 0)
        pltpu.sync_copy(out_v, d2_hbm.at[pl.ds(ebase, CHUNK)])
        return carry

    lax.fori_loop(0, N_CHUNKS, chunk_body, 0)


@functools.lru_cache(maxsize=1)
def _sc_dist2():
    mesh = plsc.VectorSubcoreMesh(
        core_axis_name="c", subcore_axis_name="s", num_cores=NC, num_subcores=NS)
    return pl.kernel(
        _sc_body,
        out_type=jax.ShapeDtypeStruct((E, 8), jnp.float32),
        mesh=mesh,
        scratch_types=[
            pltpu.VMEM((N_LIG,), jnp.float32),
            pltpu.VMEM((N_LIG,), jnp.float32),
            pltpu.VMEM((N_LIG,), jnp.float32),
            pltpu.VMEM((CHUNK,), jnp.int32),
            pltpu.VMEM((IDX_ROWS_PER_CHUNK, IDX_MINOR), jnp.int32),
            pltpu.VMEM((CHUNK, 16), jnp.float32),
            pltpu.VMEM((CHUNK, 8), jnp.float32),
            pltpu.SemaphoreType.DMA,
        ],
        compiler_params=pltpu.CompilerParams(
            needs_layout_passes=False, use_tc_tiling_on_sc=False),
    )


_COEFF = -0.5 / (PROTEIN_RADIUS / (RADIUS_EMB_DIM - 1)) ** 2


def _tc_body(d2_ref, s_ref, off_ref, w1_ref, b1_ref, w2_ref, b2_ref, o_ref):
    d2 = d2_ref[...]
    colmask = lax.broadcasted_iota(jnp.int32, (BLK, 8), 1) < 5
    d = jnp.sqrt(jnp.where(colmask, d2, 0.0) + 1e-12)
    dd = jnp.dot(d, s_ref[...], preferred_element_type=jnp.float32,
                 precision=lax.Precision.HIGHEST)
    t = dd - off_ref[...]
    ea = jnp.exp(_COEFF * (t * t))
    h = jnp.maximum(
        jnp.dot(ea, w1_ref[...], preferred_element_type=jnp.float32) + b1_ref[...], 0.0)
    o_ref[...] = jnp.dot(h, w2_ref[...], preferred_element_type=jnp.float32) + b2_ref[...]


def kernel(lig_pos, prot_pos, prot_pos_Cb, prot_pos_C, prot_pos_O, prot_pos_N,
           cross_idx, W1, b1, W2, b2):
    p_packed = jnp.concatenate(
        [prot_pos, prot_pos_Cb, prot_pos_C, prot_pos_O, prot_pos_N,
         jnp.zeros((N_PROT, 1), jnp.float32)], axis=1)
    src = cross_idx[0].astype(jnp.int32)
    dst2d = cross_idx[1].astype(jnp.int32).reshape(E // IDX_MINOR, IDX_MINOR)
    ligx = lig_pos[:, 0]
    ligy = lig_pos[:, 1]
    ligz = lig_pos[:, 2]

    d2 = _sc_dist2()(p_packed, ligx, ligy, ligz, src, dst2d)

    offset = jnp.linspace(0.0, PROTEIN_RADIUS, RADIUS_EMB_DIM)
    off_full = jnp.concatenate(
        [jnp.tile(offset, 5), jnp.zeros((FEAT_PAD - FEAT,), jnp.float32)])[None, :]
    sel = (jnp.arange(FEAT_PAD)[None, :] // RADIUS_EMB_DIM) == jnp.arange(8)[:, None]
    sel = sel & (jnp.arange(FEAT_PAD)[None, :] < FEAT)
    s_mat = sel.astype(jnp.float32)
    w1p = jnp.concatenate(
        [W1, jnp.zeros((FEAT_PAD - FEAT, FOLD_DIM), jnp.float32)], axis=0)

    out = pl.pallas_call(
        _tc_body,
        grid=(E // BLK,),
        in_specs=[
            pl.BlockSpec((BLK, 8), lambda i: (i, 0)),
            pl.BlockSpec((8, FEAT_PAD), lambda i: (0, 0)),
            pl.BlockSpec((1, FEAT_PAD), lambda i: (0, 0)),
            pl.BlockSpec((FEAT_PAD, FOLD_DIM), lambda i: (0, 0)),
            pl.BlockSpec((1, FOLD_DIM), lambda i: (0, 0)),
            pl.BlockSpec((FOLD_DIM, FOLD_DIM), lambda i: (0, 0)),
            pl.BlockSpec((1, FOLD_DIM), lambda i: (0, 0)),
        ],
        out_specs=pl.BlockSpec((BLK, FOLD_DIM), lambda i: (i, 0)),
        out_shape=jax.ShapeDtypeStruct((E, FOLD_DIM), jnp.float32),
    )(d2, s_mat, off_full, w1p, b1[None, :], W2, b2[None, :])
    return out
